# baseline (speedup 1.0000x reference)
import jax
import jax.numpy as jnp
from jax import lax
from jax.experimental import pallas as pl
from jax.experimental.pallas import tpu as pltpu
from jax.experimental.pallas import tpu_sc as plsc

NC = 2
NS = 16
NW = NC * NS
L = 16

N_ATOMS = 2000000
ROWS = N_ATOMS // 128
BROWS = 1024
NBLK = (ROWS + BROWS - 1) // BROWS
PROWS = NBLK * BROWS
CROWS = 64
NCHUNK = PROWS // CROWS
ITERS = NCHUNK // NW
NBINS = 64
ACC = NBINS * L


def _fmt_body(ad_ref, cr_ref, al_ref,
              key_ref, c0_ref, c1_ref, c2_ref):
    i = pl.program_id(0)
    ad = ad_ref[...]
    al = al_ref[...].astype(jnp.int32)
    row = i * BROWS + lax.broadcasted_iota(jnp.int32, (BROWS, 128), 0)
    valid = row < ROWS
    bits = jnp.where(valid, al[0] + 2 * al[1] + 4 * al[2], 0)
    key_ref[...] = ((ad[3] * 4 + ad[2]) * 4 + ad[1]
                    + jnp.where(ad[0] < 2, NBINS, 0)
                    + bits * 128)
    cr = cr_ref[...]
    c0_ref[...] = cr[0]
    c1_ref[...] = cr[1]
    c2_ref[...] = cr[2]


def _sc_body(key_hbm, c0_hbm, c1_hbm, c2_hbm, out_hbm,
             key_v, c0_v, c1_v, c2_v,
             mc0, mc1, mc2, sc0, sc1, sc2, ct0, ct1, ct2, res_v,
             sem0, sem1):
    accs = (mc0, mc1, mc2, sc0, sc1, sc2, ct0, ct1, ct2)
    hbms = (key_hbm, c0_hbm, c1_hbm, c2_hbm)
    bufs = (key_v, c0_v, c1_v, c2_v)
    sems = (sem0, sem1)
    cid = lax.axis_index("c")
    sid = lax.axis_index("s")
    wid = sid * NC + cid
    lane = lax.iota(jnp.int32, L)
    lane_off = lane * NBINS
    zeros = jnp.zeros((L,), jnp.float32)
    ones = jnp.ones((L,), jnp.float32)
    izeros = jnp.zeros((L,), jnp.int32)
    cols = [q * L + lane for q in range(128 // L)]

    for a in accs:
        for q in range(ACC // L):
            a[pl.ds(q * L, L)] = zeros

    def start(b, c):
        for h, v in zip(hbms, bufs):
            pltpu.async_copy(h.at[pl.ds(c * CROWS, CROWS)], v.at[b], sems[b])

    def wait(b, c):
        for h, v in zip(hbms, bufs):
            pltpu.make_async_copy(h.at[pl.ds(c * CROWS, CROWS)], v.at[b],
                                  sems[b]).wait()

    def make_row_body(b):
        crs = (c0_v.at[b], c1_v.at[b], c2_v.at[b])
        keyr = key_v.at[b]

        def row_body(r, carry):
            rowv = izeros + r
            for q in range(128 // L):
                col = cols[q]
                key = plsc.load_gather(keyr, [rowv, col])
                binv = key & 63
                slot = lane_off + binv
                bb = (key & NBINS) == NBINS
                nbb = (key & NBINS) == 0
                for alt in range(3):
                    cont = plsc.load_gather(crs[alt], [rowv, col])
                    alive = (lax.shift_right_logical(key, 7 + alt) & 1) == 1
                    m_mc = alive & bb
                    m_sc = alive & nbb
                    plsc.addupdate_scatter(accs[alt], [slot], cont,
                                           mask=m_mc)
                    plsc.addupdate_scatter(accs[3 + alt], [slot], cont,
                                           mask=m_sc)
                    plsc.addupdate_scatter(accs[6 + alt], [slot], ones,
                                           mask=m_mc)
            return carry

        return row_body

    rb0 = make_row_body(0)
    rb1 = make_row_body(1)

    start(0, wid)

    def pair_body(i, carry):
        c0 = wid + (2 * i) * NW
        c1 = wid + (2 * i + 1) * NW
        start(1, c1)
        wait(0, c0)
        lax.fori_loop(0, CROWS, rb0, 0)

        @pl.when(i + 1 < ITERS // 2)
        def _():
            start(0, wid + (2 * i + 2) * NW)
        wait(1, c1)
        lax.fori_loop(0, CROWS, rb1, 0)
        return carry

    lax.fori_loop(0, ITERS // 2, pair_body, 0)

    for k in range(9):
        a = accs[k]
        for q in range(NBINS // L):
            s = a[pl.ds(q * L, L)]
            for r in range(1, L):
                s = s + a[pl.ds(r * NBINS + q * L, L)]
            res_v[pl.ds(k * NBINS + q * L, L)] = s
    pltpu.sync_copy(res_v, out_hbm.at[wid])


def _combine_body(p_ref, mc_ref, sc_ref):
    s = jnp.sum(p_ref[...], axis=0)
    mc = s[0:3]
    sc = s[3:6]
    written = s[6:9] > 0.0
    mc_ref[...] = jnp.clip(jnp.where(written, (mc - 2.0) / 38.0, mc), 0.0, 1.0)
    sc_ref[...] = jnp.clip(jnp.where(written, (sc - 5.0) / 95.0, sc), 0.0, 1.0)


def kernel(contRat, atom_description, alternatives):
    adT = atom_description.astype(jnp.int32).T.reshape(5, ROWS, 128)
    crT = contRat.T.reshape(3, ROWS, 128)
    alT = alternatives.T.reshape(3, ROWS, 128)

    sds = jax.ShapeDtypeStruct
    key2, c02, c12, c22 = pl.pallas_call(
        _fmt_body,
        grid=(NBLK,),
        in_specs=[
            pl.BlockSpec((5, BROWS, 128), lambda i: (0, i, 0)),
            pl.BlockSpec((3, BROWS, 128), lambda i: (0, i, 0)),
            pl.BlockSpec((3, BROWS, 128), lambda i: (0, i, 0)),
        ],
        out_specs=[pl.BlockSpec((BROWS, 128), lambda i: (i, 0))] * 4,
        out_shape=[sds((PROWS, 128), jnp.int32),
                   sds((PROWS, 128), jnp.float32),
                   sds((PROWS, 128), jnp.float32),
                   sds((PROWS, 128), jnp.float32)],
    )(adT, crT, alT)

    mesh = plsc.VectorSubcoreMesh(core_axis_name="c", subcore_axis_name="s")
    scratch = [
        pltpu.VMEM((2, CROWS, 128), jnp.int32),
        pltpu.VMEM((2, CROWS, 128), jnp.float32),
        pltpu.VMEM((2, CROWS, 128), jnp.float32),
        pltpu.VMEM((2, CROWS, 128), jnp.float32),
    ] + [pltpu.VMEM((ACC,), jnp.float32) for _ in range(9)] + [
        pltpu.VMEM((9 * NBINS,), jnp.float32),
        pltpu.SemaphoreType.DMA,
        pltpu.SemaphoreType.DMA,
    ]
    partials = pl.kernel(
        _sc_body,
        out_type=jax.ShapeDtypeStruct((NW, 9 * NBINS), jnp.float32),
        mesh=mesh,
        scratch_types=scratch,
        compiler_params=pltpu.CompilerParams(needs_layout_passes=False),
    )(key2, c02, c12, c22)

    mcn, scn = pl.pallas_call(
        _combine_body,
        out_shape=[jax.ShapeDtypeStruct((3, NBINS), jnp.float32),
                   jax.ShapeDtypeStruct((3, NBINS), jnp.float32)],
    )(partials.reshape(NW, 9, NBINS))
    rsaMC = mcn.T.reshape(4, 4, 4, 3)
    rsaSC = scn.T.reshape(4, 4, 4, 3)
    return rsaMC, rsaSC
